# baseline (speedup 1.0000x reference)
import functools

import jax
import jax.numpy as jnp
from jax.experimental import pallas as pl
from jax.experimental.pallas import tpu as pltpu

_SCALES = (0.25, 0.177, 0.1255)
_SIZES = (128, 90, 64)
_K = 100
_THRESH = 0.6
_NMS_T = 0.7
_NS = tuple(((s - 2) // 2 - 4) ** 2 for s in _SIZES)
_NTOT = sum(_NS)
_NPAD = 46 * 128


def _conv3x3(x, wmat, b, a):
    C, H, W = x.shape
    Ho, Wo = H - 2, W - 2
    win = jnp.concatenate(
        [x[:, di:di + Ho, dj:dj + Wo] for di in range(3) for dj in range(3)],
        axis=0)
    y = jax.lax.dot_general(
        wmat, win, (((1,), (0,)), ((), ())),
        preferred_element_type=jnp.float32, precision=jax.lax.Precision.DEFAULT)
    y = y + b
    return jnp.where(y > 0, y, a * y)


def _pnet_scale_kernel(sc, s2,
                       img_ref, r_ref, w1_ref, b1_ref, a1_ref, w2_ref, b2_ref,
                       a2_ref, w3_ref, b3_ref, a3_ref, w41_ref, b41_ref,
                       w42_ref, b42_ref, score_ref, box_ref):
    x = (img_ref[0] - 127.5) * (1.0 / 128.0)
    r = r_ref[...]
    t = jax.lax.dot_general(x, r, (((1,), (1,)), ((), ())),
                            preferred_element_type=jnp.float32, precision=jax.lax.Precision.HIGHEST)
    im = jax.lax.dot_general(t, r, (((1,), (1,)), ((), ())),
                             preferred_element_type=jnp.float32, precision=jax.lax.Precision.HIGHEST)

    y1 = _conv3x3(im, w1_ref[...], b1_ref[...], a1_ref[...])
    C1, H1, W1 = y1.shape
    Hp, Wp = H1 // 2, W1 // 2
    a4 = y1.reshape(C1, Hp, 2, W1)
    m = jnp.maximum(a4[:, :, 0, :], a4[:, :, 1, :])
    rpair = jnp.maximum(m[:, :, :W1 - 1], m[:, :, 1:])
    dsel = (jax.lax.broadcasted_iota(jnp.int32, (W1 - 1, Wp), 0)
            == 2 * jax.lax.broadcasted_iota(jnp.int32, (W1 - 1, Wp), 1)
            ).astype(jnp.float32)
    p = jax.lax.dot_general(rpair, dsel, (((2,), (0,)), ((), ())),
                            preferred_element_type=jnp.float32, precision=jax.lax.Precision.HIGHEST)

    y2 = _conv3x3(p, w2_ref[...], b2_ref[...], a2_ref[...])
    y3 = _conv3x3(y2, w3_ref[...], b3_ref[...], a3_ref[...])

    logits = jax.lax.dot_general(w41_ref[...], y3, (((1,), (0,)), ((), ())),
                                 preferred_element_type=jnp.float32, precision=jax.lax.Precision.DEFAULT)
    logits = logits + b41_ref[...]
    prob = jax.nn.sigmoid(logits[1] - logits[0])
    reg = jax.lax.dot_general(w42_ref[...], y3, (((1,), (0,)), ((), ())),
                              preferred_element_type=jnp.float32, precision=jax.lax.Precision.DEFAULT)
    reg = reg + b42_ref[...]

    H3, W3 = prob.shape
    gy = jax.lax.broadcasted_iota(jnp.int32, (H3, W3), 0).astype(jnp.float32)
    gx = jax.lax.broadcasted_iota(jnp.int32, (H3, W3), 1).astype(jnp.float32)
    inv = 1.0 / sc
    x1 = (2.0 * gx + 1.0) * inv
    yy1 = (2.0 * gy + 1.0) * inv
    x2 = (2.0 * gx + 12.0) * inv
    yy2 = (2.0 * gy + 12.0) * inv
    bw = x2 - x1
    bh = yy2 - yy1
    bx1 = x1 + reg[0] * bw
    by1 = yy1 + reg[1] * bh
    bx2 = x2 + reg[2] * bw
    by2 = yy2 + reg[3] * bh

    score_ref[0] = prob
    box_ref[0] = jnp.stack([bx1, by1, bx2, by2], axis=0)


def _run_pnet_scale(idx, img, r, w1m, b1, a1, w2m, b2, a2, w3m, b3, a3,
                    w41m, b41, w42m, b42):
    s2 = _SIZES[idx]
    h3 = (s2 - 2) // 2 - 4
    B = img.shape[0]
    full = lambda arr: pl.BlockSpec(arr.shape, lambda b: (0,) * arr.ndim)
    kern = functools.partial(_pnet_scale_kernel, _SCALES[idx], s2)
    score, box = pl.pallas_call(
        kern,
        grid=(B,),
        in_specs=[
            pl.BlockSpec((1, 3, 512, 512), lambda b: (b, 0, 0, 0)),
            full(r), full(w1m), full(b1), full(a1), full(w2m), full(b2),
            full(a2), full(w3m), full(b3), full(a3), full(w41m), full(b41),
            full(w42m), full(b42),
        ],
        out_specs=[
            pl.BlockSpec((1, h3, h3), lambda b: (b, 0, 0)),
            pl.BlockSpec((1, 4, h3, h3), lambda b: (b, 0, 0, 0)),
        ],
        out_shape=[
            jax.ShapeDtypeStruct((B, h3, h3), jnp.float32),
            jax.ShapeDtypeStruct((B, 4, h3, h3), jnp.float32),
        ],
        name=f"pnet_s{idx}",
        compiler_params=pltpu.CompilerParams(dimension_semantics=("parallel",)),
    )(img, r, w1m, b1, a1, w2m, b2, a2, w3m, b3, a3, w41m, b41, w42m, b42)
    return score, box


def _topk_nms_kernel(s_ref, bx_ref, out_ref):
    S = s_ref[0]
    BX = bx_ref[0]
    shape = S.shape
    pos = (jax.lax.broadcasted_iota(jnp.int32, shape, 0) * 128
           + jax.lax.broadcasted_iota(jnp.int32, shape, 1))
    posf = pos.astype(jnp.float32)
    S = jnp.where(pos < _NTOT, S, -1e30)
    X1, Y1, X2, Y2 = BX[0], BX[1], BX[2], BX[3]
    AREA = (X2 - X1) * (Y2 - Y1)
    lane = jax.lax.broadcasted_iota(jnp.int32, (1, 128), 1)
    z128 = jnp.zeros((1, 128), jnp.float32)

    def body(k, carry):
        Sc, kx1, ky1, kx2, ky2, ks, karea, kkeep = carry
        mflat = jnp.max(Sc)
        hit = Sc == mflat
        selp = jnp.min(jnp.where(hit, posf, 1e9))
        hsel = posf == selp
        hself = hsel.astype(jnp.float32)
        bx1 = jnp.sum(hself * X1)
        by1 = jnp.sum(hself * Y1)
        bx2 = jnp.sum(hself * X2)
        by2 = jnp.sum(hself * Y2)
        barea = jnp.sum(hself * AREA)
        xx1 = jnp.maximum(bx1, kx1)
        yy1 = jnp.maximum(by1, ky1)
        xx2 = jnp.minimum(bx2, kx2)
        yy2 = jnp.minimum(by2, ky2)
        inter = jnp.maximum(0.0, xx2 - xx1) * jnp.maximum(0.0, yy2 - yy1)
        iou = inter / (barea + karea - inter + 1e-9)
        sup = jnp.max(jnp.where((kkeep > 0) & (iou > _NMS_T), 1.0, 0.0))
        keep_k = jnp.where((mflat > _THRESH) & (sup == 0), 1.0, 0.0)
        lhot = lane == k
        kx1 = jnp.where(lhot, bx1, kx1)
        ky1 = jnp.where(lhot, by1, ky1)
        kx2 = jnp.where(lhot, bx2, kx2)
        ky2 = jnp.where(lhot, by2, ky2)
        ks = jnp.where(lhot, mflat, ks)
        karea = jnp.where(lhot, barea, karea)
        kkeep = jnp.where(lhot, keep_k, kkeep)
        Sc = jnp.where(hsel, -1e30, Sc)
        return Sc, kx1, ky1, kx2, ky2, ks, karea, kkeep

    init = (S, z128, z128, z128, z128, z128, z128, z128)
    _, kx1, ky1, kx2, ky2, ks, _, kkeep = jax.lax.fori_loop(0, _K, body, init)
    out_ref[0] = jnp.concatenate(
        [kx1, ky1, kx2, ky2, ks * kkeep, jnp.zeros((3, 128), jnp.float32)],
        axis=0)


@functools.lru_cache(maxsize=1)
def _resize_mats():
    eye = jnp.eye(512, dtype=jnp.float32)
    return tuple(jax.image.resize(eye, (s2, 512), method='bilinear')
                 for s2 in _SIZES)


def kernel(img, w1, b1, a1, w2, b2, a2, w3, b3, a3, w41, b41, w42, b42):
    B = img.shape[0]
    rs = _resize_mats()

    w1m = w1.transpose(0, 2, 3, 1).reshape(10, 27)
    w2m = w2.transpose(0, 2, 3, 1).reshape(16, 90)
    w3m = w3.transpose(0, 2, 3, 1).reshape(32, 144)
    w41m = w41.reshape(2, 32)
    w42m = w42.reshape(4, 32)
    b1r, a1r = b1.reshape(10, 1, 1), a1.reshape(10, 1, 1)
    b2r, a2r = b2.reshape(16, 1, 1), a2.reshape(16, 1, 1)
    b3r, a3r = b3.reshape(32, 1, 1), a3.reshape(32, 1, 1)
    b41r = b41.reshape(2, 1, 1)
    b42r = b42.reshape(4, 1, 1)

    scores = []
    boxes = []
    for i in range(3):
        s, bx = _run_pnet_scale(i, img, rs[i], w1m, b1r, a1r, w2m, b2r, a2r,
                                w3m, b3r, a3r, w41m, b41r, w42m, b42r)
        scores.append(s.reshape(B, -1))
        boxes.append(bx.reshape(B, 4, -1))

    s_all = jnp.concatenate(scores, axis=1)
    b_all = jnp.concatenate(boxes, axis=2)
    s_pl = jnp.pad(s_all, ((0, 0), (0, _NPAD - _NTOT))).reshape(B, 46, 128)
    b_pl = jnp.pad(b_all, ((0, 0), (0, 0), (0, _NPAD - _NTOT))).reshape(B, 4, 46, 128)

    out = pl.pallas_call(
        _topk_nms_kernel,
        grid=(B,),
        in_specs=[
            pl.BlockSpec((1, 46, 128), lambda b: (b, 0, 0)),
            pl.BlockSpec((1, 4, 46, 128), lambda b: (b, 0, 0, 0)),
        ],
        out_specs=pl.BlockSpec((1, 8, 128), lambda b: (b, 0, 0)),
        out_shape=jax.ShapeDtypeStruct((B, 8, 128), jnp.float32),
        name="topk_nms",
        compiler_params=pltpu.CompilerParams(dimension_semantics=("parallel",)),
    )(s_pl, b_pl)

    res = out[:, :5, :_K].transpose(0, 2, 1)
    return res


# speedup vs baseline: 1.1137x; 1.1137x over previous
import functools

import jax
import jax.numpy as jnp
from jax.experimental import pallas as pl
from jax.experimental.pallas import tpu as pltpu

_SCALES = (0.25, 0.177, 0.1255)
_SIZES = (128, 90, 64)
_K = 100
_THRESH = 0.6
_NMS_T = 0.7
_NS = tuple(((s - 2) // 2 - 4) ** 2 for s in _SIZES)
_NTOT = sum(_NS)
_NPAD = 46 * 128


def _conv3x3(x, wmat, b, a):
    C, H, W = x.shape
    Ho, Wo = H - 2, W - 2
    win = jnp.concatenate(
        [x[:, di:di + Ho, dj:dj + Wo] for di in range(3) for dj in range(3)],
        axis=0)
    y = jax.lax.dot_general(
        wmat, win, (((1,), (0,)), ((), ())),
        preferred_element_type=jnp.float32, precision=jax.lax.Precision.DEFAULT)
    y = y + b
    return jnp.where(y > 0, y, a * y)


def _pnet_scale_kernel(sc, s2,
                       img_ref, r_ref, w1_ref, b1_ref, a1_ref, w2_ref, b2_ref,
                       a2_ref, w3_ref, b3_ref, a3_ref, w41_ref, b41_ref,
                       w42_ref, b42_ref, score_ref, box_ref):
    x = (img_ref[0] - 127.5) * (1.0 / 128.0)
    r = r_ref[...]
    t = jax.lax.dot_general(x, r, (((1,), (1,)), ((), ())),
                            preferred_element_type=jnp.float32, precision=jax.lax.Precision.HIGHEST)
    im = jax.lax.dot_general(t, r, (((1,), (1,)), ((), ())),
                             preferred_element_type=jnp.float32, precision=jax.lax.Precision.HIGHEST)

    y1 = _conv3x3(im, w1_ref[...], b1_ref[...], a1_ref[...])
    C1, H1, W1 = y1.shape
    Hp, Wp = H1 // 2, W1 // 2
    a4 = y1.reshape(C1, Hp, 2, W1)
    m = jnp.maximum(a4[:, :, 0, :], a4[:, :, 1, :])
    rpair = jnp.maximum(m[:, :, :W1 - 1], m[:, :, 1:])
    dsel = (jax.lax.broadcasted_iota(jnp.int32, (W1 - 1, Wp), 0)
            == 2 * jax.lax.broadcasted_iota(jnp.int32, (W1 - 1, Wp), 1)
            ).astype(jnp.float32)
    p = jax.lax.dot_general(rpair, dsel, (((2,), (0,)), ((), ())),
                            preferred_element_type=jnp.float32, precision=jax.lax.Precision.HIGHEST)

    y2 = _conv3x3(p, w2_ref[...], b2_ref[...], a2_ref[...])
    y3 = _conv3x3(y2, w3_ref[...], b3_ref[...], a3_ref[...])

    logits = jax.lax.dot_general(w41_ref[...], y3, (((1,), (0,)), ((), ())),
                                 preferred_element_type=jnp.float32, precision=jax.lax.Precision.DEFAULT)
    logits = logits + b41_ref[...]
    prob = jax.nn.sigmoid(logits[1] - logits[0])
    reg = jax.lax.dot_general(w42_ref[...], y3, (((1,), (0,)), ((), ())),
                              preferred_element_type=jnp.float32, precision=jax.lax.Precision.DEFAULT)
    reg = reg + b42_ref[...]

    H3, W3 = prob.shape
    gy = jax.lax.broadcasted_iota(jnp.int32, (H3, W3), 0).astype(jnp.float32)
    gx = jax.lax.broadcasted_iota(jnp.int32, (H3, W3), 1).astype(jnp.float32)
    inv = 1.0 / sc
    x1 = (2.0 * gx + 1.0) * inv
    yy1 = (2.0 * gy + 1.0) * inv
    x2 = (2.0 * gx + 12.0) * inv
    yy2 = (2.0 * gy + 12.0) * inv
    bw = x2 - x1
    bh = yy2 - yy1
    bx1 = x1 + reg[0] * bw
    by1 = yy1 + reg[1] * bh
    bx2 = x2 + reg[2] * bw
    by2 = yy2 + reg[3] * bh

    score_ref[0] = prob
    box_ref[0] = jnp.stack([bx1, by1, bx2, by2], axis=0)


def _run_pnet_scale(idx, img, r, w1m, b1, a1, w2m, b2, a2, w3m, b3, a3,
                    w41m, b41, w42m, b42):
    s2 = _SIZES[idx]
    h3 = (s2 - 2) // 2 - 4
    B = img.shape[0]
    full = lambda arr: pl.BlockSpec(arr.shape, lambda b: (0,) * arr.ndim)
    kern = functools.partial(_pnet_scale_kernel, _SCALES[idx], s2)
    score, box = pl.pallas_call(
        kern,
        grid=(B,),
        in_specs=[
            pl.BlockSpec((1, 3, 512, 512), lambda b: (b, 0, 0, 0)),
            full(r), full(w1m), full(b1), full(a1), full(w2m), full(b2),
            full(a2), full(w3m), full(b3), full(a3), full(w41m), full(b41),
            full(w42m), full(b42),
        ],
        out_specs=[
            pl.BlockSpec((1, h3, h3), lambda b: (b, 0, 0)),
            pl.BlockSpec((1, 4, h3, h3), lambda b: (b, 0, 0, 0)),
        ],
        out_shape=[
            jax.ShapeDtypeStruct((B, h3, h3), jnp.float32),
            jax.ShapeDtypeStruct((B, 4, h3, h3), jnp.float32),
        ],
        name=f"pnet_s{idx}",
        compiler_params=pltpu.CompilerParams(dimension_semantics=("parallel",)),
    )(img, r, w1m, b1, a1, w2m, b2, a2, w3m, b3, a3, w41m, b41, w42m, b42)
    return score, box


def _topk_nms_kernel(s_ref, bx_ref, out_ref):
    B = s_ref.shape[0]
    shape = (46, 128)
    pos = (jax.lax.broadcasted_iota(jnp.int32, shape, 0) * 128
           + jax.lax.broadcasted_iota(jnp.int32, shape, 1))
    posf = pos.astype(jnp.float32)
    lane = jax.lax.broadcasted_iota(jnp.int32, (1, 128), 1)
    z128 = jnp.zeros((1, 128), jnp.float32)

    init = []
    for b in range(B):
        S = jnp.where(pos < _NTOT, s_ref[b], -1e30)
        init.append((S, z128, z128, z128, z128, z128, z128, z128))
    init = tuple(x for st in init for x in st)

    def body(k, carry):
        sts = [carry[8 * b:8 * b + 8] for b in range(B)]
        out = []
        for b in range(B):
            Sc, kx1, ky1, kx2, ky2, ks, karea, kkeep = sts[b]
            X1 = bx_ref[b, 0]
            Y1 = bx_ref[b, 1]
            X2 = bx_ref[b, 2]
            Y2 = bx_ref[b, 3]
            AREA = (X2 - X1) * (Y2 - Y1)
            mflat = jnp.max(Sc)
            hit = Sc == mflat
            selp = jnp.min(jnp.where(hit, posf, 1e9))
            hsel = posf == selp
            hself = hsel.astype(jnp.float32)
            bx1 = jnp.sum(hself * X1)
            by1 = jnp.sum(hself * Y1)
            bx2 = jnp.sum(hself * X2)
            by2 = jnp.sum(hself * Y2)
            barea = jnp.sum(hself * AREA)
            xx1 = jnp.maximum(bx1, kx1)
            yy1 = jnp.maximum(by1, ky1)
            xx2 = jnp.minimum(bx2, kx2)
            yy2 = jnp.minimum(by2, ky2)
            inter = jnp.maximum(0.0, xx2 - xx1) * jnp.maximum(0.0, yy2 - yy1)
            iou = inter / (barea + karea - inter + 1e-9)
            sup = jnp.max(jnp.where((kkeep > 0) & (iou > _NMS_T), 1.0, 0.0))
            keep_k = jnp.where((mflat > _THRESH) & (sup == 0), 1.0, 0.0)
            lhot = lane == k
            kx1 = jnp.where(lhot, bx1, kx1)
            ky1 = jnp.where(lhot, by1, ky1)
            kx2 = jnp.where(lhot, bx2, kx2)
            ky2 = jnp.where(lhot, by2, ky2)
            ks = jnp.where(lhot, mflat, ks)
            karea = jnp.where(lhot, barea, karea)
            kkeep = jnp.where(lhot, keep_k, kkeep)
            Sc = jnp.where(hsel, -1e30, Sc)
            out.append((Sc, kx1, ky1, kx2, ky2, ks, karea, kkeep))
        return tuple(x for st in out for x in st)

    fin = jax.lax.fori_loop(0, _K, body, init)
    for b in range(B):
        _, kx1, ky1, kx2, ky2, ks, _, kkeep = fin[8 * b:8 * b + 8]
        out_ref[b] = jnp.concatenate(
            [kx1, ky1, kx2, ky2, ks * kkeep, jnp.zeros((3, 128), jnp.float32)],
            axis=0)


@functools.lru_cache(maxsize=1)
def _resize_mats():
    eye = jnp.eye(512, dtype=jnp.float32)
    return tuple(jax.image.resize(eye, (s2, 512), method='bilinear')
                 for s2 in _SIZES)


def kernel(img, w1, b1, a1, w2, b2, a2, w3, b3, a3, w41, b41, w42, b42):
    B = img.shape[0]
    rs = _resize_mats()

    w1m = w1.transpose(0, 2, 3, 1).reshape(10, 27)
    w2m = w2.transpose(0, 2, 3, 1).reshape(16, 90)
    w3m = w3.transpose(0, 2, 3, 1).reshape(32, 144)
    w41m = w41.reshape(2, 32)
    w42m = w42.reshape(4, 32)
    b1r, a1r = b1.reshape(10, 1, 1), a1.reshape(10, 1, 1)
    b2r, a2r = b2.reshape(16, 1, 1), a2.reshape(16, 1, 1)
    b3r, a3r = b3.reshape(32, 1, 1), a3.reshape(32, 1, 1)
    b41r = b41.reshape(2, 1, 1)
    b42r = b42.reshape(4, 1, 1)

    scores = []
    boxes = []
    for i in range(3):
        s, bx = _run_pnet_scale(i, img, rs[i], w1m, b1r, a1r, w2m, b2r, a2r,
                                w3m, b3r, a3r, w41m, b41r, w42m, b42r)
        scores.append(s.reshape(B, -1))
        boxes.append(bx.reshape(B, 4, -1))

    s_all = jnp.concatenate(scores, axis=1)
    b_all = jnp.concatenate(boxes, axis=2)
    s_pl = jnp.pad(s_all, ((0, 0), (0, _NPAD - _NTOT))).reshape(B, 46, 128)
    b_pl = jnp.pad(b_all, ((0, 0), (0, 0), (0, _NPAD - _NTOT))).reshape(B, 4, 46, 128)

    out = pl.pallas_call(
        _topk_nms_kernel,
        out_shape=jax.ShapeDtypeStruct((B, 8, 128), jnp.float32),
        name="topk_nms",
    )(s_pl, b_pl)

    res = out[:, :5, :_K].transpose(0, 2, 1)
    return res


# speedup vs baseline: 1.2542x; 1.1261x over previous
import functools

import jax
import jax.numpy as jnp
from jax.experimental import pallas as pl
from jax.experimental.pallas import tpu as pltpu

_SCALES = (0.25, 0.177, 0.1255)
_SIZES = (128, 90, 64)
_K = 100
_THRESH = 0.6
_NMS_T = 0.7
_NS = tuple(((s - 2) // 2 - 4) ** 2 for s in _SIZES)
_NTOT = sum(_NS)
_NPAD = 46 * 128


def _conv3x3(x, wmat, b, a):
    C, H, W = x.shape
    Ho, Wo = H - 2, W - 2
    win = jnp.concatenate(
        [x[:, di:di + Ho, dj:dj + Wo] for di in range(3) for dj in range(3)],
        axis=0)
    y = jax.lax.dot_general(
        wmat, win, (((1,), (0,)), ((), ())),
        preferred_element_type=jnp.float32, precision=jax.lax.Precision.DEFAULT)
    y = y + b
    return jnp.where(y > 0, y, a * y)


def _pnet_scale_kernel(sc, s2,
                       img_ref, r_ref, w1_ref, b1_ref, a1_ref, w2_ref, b2_ref,
                       a2_ref, w3_ref, b3_ref, a3_ref, w41_ref, b41_ref,
                       w42_ref, b42_ref, score_ref, box_ref):
    x = (img_ref[0] - 127.5) * (1.0 / 128.0)
    r = r_ref[...]
    t = jax.lax.dot_general(x, r, (((1,), (1,)), ((), ())),
                            preferred_element_type=jnp.float32, precision=jax.lax.Precision.HIGHEST)
    im = jax.lax.dot_general(t, r, (((1,), (1,)), ((), ())),
                             preferred_element_type=jnp.float32, precision=jax.lax.Precision.HIGHEST)

    y1 = _conv3x3(im, w1_ref[...], b1_ref[...], a1_ref[...])
    C1, H1, W1 = y1.shape
    Hp, Wp = H1 // 2, W1 // 2
    a4 = y1.reshape(C1, Hp, 2, W1)
    m = jnp.maximum(a4[:, :, 0, :], a4[:, :, 1, :])
    rpair = jnp.maximum(m[:, :, :W1 - 1], m[:, :, 1:])
    dsel = (jax.lax.broadcasted_iota(jnp.int32, (W1 - 1, Wp), 0)
            == 2 * jax.lax.broadcasted_iota(jnp.int32, (W1 - 1, Wp), 1)
            ).astype(jnp.float32)
    p = jax.lax.dot_general(rpair, dsel, (((2,), (0,)), ((), ())),
                            preferred_element_type=jnp.float32, precision=jax.lax.Precision.HIGHEST)

    y2 = _conv3x3(p, w2_ref[...], b2_ref[...], a2_ref[...])
    y3 = _conv3x3(y2, w3_ref[...], b3_ref[...], a3_ref[...])

    logits = jax.lax.dot_general(w41_ref[...], y3, (((1,), (0,)), ((), ())),
                                 preferred_element_type=jnp.float32, precision=jax.lax.Precision.DEFAULT)
    logits = logits + b41_ref[...]
    prob = jax.nn.sigmoid(logits[1] - logits[0])
    reg = jax.lax.dot_general(w42_ref[...], y3, (((1,), (0,)), ((), ())),
                              preferred_element_type=jnp.float32, precision=jax.lax.Precision.DEFAULT)
    reg = reg + b42_ref[...]

    H3, W3 = prob.shape
    gy = jax.lax.broadcasted_iota(jnp.int32, (H3, W3), 0).astype(jnp.float32)
    gx = jax.lax.broadcasted_iota(jnp.int32, (H3, W3), 1).astype(jnp.float32)
    inv = 1.0 / sc
    x1 = (2.0 * gx + 1.0) * inv
    yy1 = (2.0 * gy + 1.0) * inv
    x2 = (2.0 * gx + 12.0) * inv
    yy2 = (2.0 * gy + 12.0) * inv
    bw = x2 - x1
    bh = yy2 - yy1
    bx1 = x1 + reg[0] * bw
    by1 = yy1 + reg[1] * bh
    bx2 = x2 + reg[2] * bw
    by2 = yy2 + reg[3] * bh

    score_ref[0] = prob
    box_ref[0] = jnp.stack([bx1, by1, bx2, by2], axis=0)


def _run_pnet_scale(idx, img, r, w1m, b1, a1, w2m, b2, a2, w3m, b3, a3,
                    w41m, b41, w42m, b42):
    s2 = _SIZES[idx]
    h3 = (s2 - 2) // 2 - 4
    B = img.shape[0]
    full = lambda arr: pl.BlockSpec(arr.shape, lambda b: (0,) * arr.ndim)
    kern = functools.partial(_pnet_scale_kernel, _SCALES[idx], s2)
    score, box = pl.pallas_call(
        kern,
        grid=(B,),
        in_specs=[
            pl.BlockSpec((1, 3, 512, 512), lambda b: (b, 0, 0, 0)),
            full(r), full(w1m), full(b1), full(a1), full(w2m), full(b2),
            full(a2), full(w3m), full(b3), full(a3), full(w41m), full(b41),
            full(w42m), full(b42),
        ],
        out_specs=[
            pl.BlockSpec((1, h3, h3), lambda b: (b, 0, 0)),
            pl.BlockSpec((1, 4, h3, h3), lambda b: (b, 0, 0, 0)),
        ],
        out_shape=[
            jax.ShapeDtypeStruct((B, h3, h3), jnp.float32),
            jax.ShapeDtypeStruct((B, 4, h3, h3), jnp.float32),
        ],
        name=f"pnet_s{idx}",
        compiler_params=pltpu.CompilerParams(dimension_semantics=("parallel",)),
    )(img, r, w1m, b1, a1, w2m, b2, a2, w3m, b3, a3, w41m, b41, w42m, b42)
    return score, box


def _topk_nms_kernel(s_ref, bx_ref, out_ref):
    B = s_ref.shape[0]
    shape = (46, 128)
    pos = (jax.lax.broadcasted_iota(jnp.int32, shape, 0) * 128
           + jax.lax.broadcasted_iota(jnp.int32, shape, 1))
    posf = pos.astype(jnp.float32)
    lane = jax.lax.broadcasted_iota(jnp.int32, (1, 128), 1)
    z128 = jnp.zeros((1, 128), jnp.float32)

    init = []
    for b in range(B):
        S = jnp.where(pos < _NTOT, s_ref[b], -1e30)
        init.append((S, z128, z128, z128, z128, z128))
    init = tuple(x for st in init for x in st)

    def body(k, carry):
        sts = [carry[6 * b:6 * b + 6] for b in range(B)]
        out = []
        lhot = lane == k
        for b in range(B):
            Sc, sx1, sy1, sx2, sy2, ss = sts[b]
            mflat = jnp.max(Sc)
            hit = Sc == mflat
            selp = jnp.min(jnp.where(hit, posf, 1e9))
            hsel = posf == selp
            hself = hsel.astype(jnp.float32)
            bx1 = jnp.sum(hself * bx_ref[b, 0])
            by1 = jnp.sum(hself * bx_ref[b, 1])
            bx2 = jnp.sum(hself * bx_ref[b, 2])
            by2 = jnp.sum(hself * bx_ref[b, 3])
            sx1 = jnp.where(lhot, bx1, sx1)
            sy1 = jnp.where(lhot, by1, sy1)
            sx2 = jnp.where(lhot, bx2, sx2)
            sy2 = jnp.where(lhot, by2, sy2)
            ss = jnp.where(lhot, mflat, ss)
            Sc = jnp.where(hsel, -1e30, Sc)
            out.append((Sc, sx1, sy1, sx2, sy2, ss))
        return tuple(x for st in out for x in st)

    fin = jax.lax.fori_loop(0, _K, body, init)

    sub = jax.lax.broadcasted_iota(jnp.int32, (128, 1), 0)
    nms_init = []
    iou_mats = []
    sorted_sts = []
    for b in range(B):
        _, sx1, sy1, sx2, sy2, ss = fin[6 * b:6 * b + 6]
        sorted_sts.append((sx1, sy1, sx2, sy2, ss))
        area_r = (sx2 - sx1) * (sy2 - sy1)
        x1c = sx1.T
        y1c = sy1.T
        x2c = sx2.T
        y2c = sy2.T
        area_c = (x2c - x1c) * (y2c - y1c)
        xx1 = jnp.maximum(x1c, sx1)
        yy1 = jnp.maximum(y1c, sy1)
        xx2 = jnp.minimum(x2c, sx2)
        yy2 = jnp.minimum(y2c, sy2)
        inter = jnp.maximum(0.0, xx2 - xx1) * jnp.maximum(0.0, yy2 - yy1)
        iou = inter / (area_c + area_r - inter + 1e-9)
        cmat = (iou > _NMS_T).astype(jnp.float32)
        iou_mats.append(cmat)
        nms_init.append((z128, z128))
    nms_init = tuple(x for st in nms_init for x in st)

    def nms_body(k, carry):
        sts = [carry[2 * b:2 * b + 2] for b in range(B)]
        out = []
        lhot = lane == k
        colhot = (sub == k).astype(jnp.float32)
        for b in range(B):
            kept, supv = sts[b]
            ss = sorted_sts[b][4]
            rowk = jnp.sum(iou_mats[b] * colhot, axis=0, keepdims=True)
            kept = jnp.where(lhot & (ss > _THRESH) & (supv < 0.5), 1.0, kept)
            keptk = jnp.max(kept * lhot.astype(jnp.float32))
            supv = jnp.where((rowk > 0.5) & (keptk > 0), 1.0, supv)
            out.append((kept, supv))
        return tuple(x for st in out for x in st)

    nfin = jax.lax.fori_loop(0, _K, nms_body, nms_init)
    for b in range(B):
        sx1, sy1, sx2, sy2, ss = sorted_sts[b]
        kept = nfin[2 * b]
        out_ref[b] = jnp.concatenate(
            [sx1, sy1, sx2, sy2, ss * kept, jnp.zeros((3, 128), jnp.float32)],
            axis=0)


@functools.lru_cache(maxsize=1)
def _resize_mats():
    eye = jnp.eye(512, dtype=jnp.float32)
    return tuple(jax.image.resize(eye, (s2, 512), method='bilinear')
                 for s2 in _SIZES)


def kernel(img, w1, b1, a1, w2, b2, a2, w3, b3, a3, w41, b41, w42, b42):
    B = img.shape[0]
    rs = _resize_mats()

    w1m = w1.transpose(0, 2, 3, 1).reshape(10, 27)
    w2m = w2.transpose(0, 2, 3, 1).reshape(16, 90)
    w3m = w3.transpose(0, 2, 3, 1).reshape(32, 144)
    w41m = w41.reshape(2, 32)
    w42m = w42.reshape(4, 32)
    b1r, a1r = b1.reshape(10, 1, 1), a1.reshape(10, 1, 1)
    b2r, a2r = b2.reshape(16, 1, 1), a2.reshape(16, 1, 1)
    b3r, a3r = b3.reshape(32, 1, 1), a3.reshape(32, 1, 1)
    b41r = b41.reshape(2, 1, 1)
    b42r = b42.reshape(4, 1, 1)

    scores = []
    boxes = []
    for i in range(3):
        s, bx = _run_pnet_scale(i, img, rs[i], w1m, b1r, a1r, w2m, b2r, a2r,
                                w3m, b3r, a3r, w41m, b41r, w42m, b42r)
        scores.append(s.reshape(B, -1))
        boxes.append(bx.reshape(B, 4, -1))

    s_all = jnp.concatenate(scores, axis=1)
    b_all = jnp.concatenate(boxes, axis=2)
    s_pl = jnp.pad(s_all, ((0, 0), (0, _NPAD - _NTOT))).reshape(B, 46, 128)
    b_pl = jnp.pad(b_all, ((0, 0), (0, 0), (0, _NPAD - _NTOT))).reshape(B, 4, 46, 128)

    out = pl.pallas_call(
        _topk_nms_kernel,
        out_shape=jax.ShapeDtypeStruct((B, 8, 128), jnp.float32),
        name="topk_nms",
    )(s_pl, b_pl)

    res = out[:, :5, :_K].transpose(0, 2, 1)
    return res


# speedup vs baseline: 1.3821x; 1.1020x over previous
import functools

import jax
import jax.numpy as jnp
from jax.experimental import pallas as pl
from jax.experimental.pallas import tpu as pltpu

_SCALES = (0.25, 0.177, 0.1255)
_SIZES = (128, 90, 64)
_K = 100
_THRESH = 0.6
_NMS_T = 0.7
_NS = tuple(((s - 2) // 2 - 4) ** 2 for s in _SIZES)
_NTOT = sum(_NS)
_NPAD = 46 * 128


def _conv3x3(x, wmat, b, a):
    C, H, W = x.shape
    Ho, Wo = H - 2, W - 2
    win = jnp.concatenate(
        [x[:, di:di + Ho, dj:dj + Wo] for di in range(3) for dj in range(3)],
        axis=0)
    y = jax.lax.dot_general(
        wmat, win, (((1,), (0,)), ((), ())),
        preferred_element_type=jnp.float32, precision=jax.lax.Precision.DEFAULT)
    y = y + b
    return jnp.where(y > 0, y, a * y)


def _pnet_scale_kernel(sc, s2,
                       img_ref, w1_ref, b1_ref, a1_ref, w2_ref, b2_ref,
                       a2_ref, w3_ref, b3_ref, a3_ref, w41_ref, b41_ref,
                       w42_ref, b42_ref, score_ref, box_ref):
    im = img_ref[0]

    y1 = _conv3x3(im, w1_ref[...], b1_ref[...], a1_ref[...])
    C1, H1, W1 = y1.shape
    Hp, Wp = H1 // 2, W1 // 2
    a4 = y1.reshape(C1, Hp, 2, W1)
    m = jnp.maximum(a4[:, :, 0, :], a4[:, :, 1, :])
    rpair = jnp.maximum(m[:, :, :W1 - 1], m[:, :, 1:])
    dsel = (jax.lax.broadcasted_iota(jnp.int32, (W1 - 1, Wp), 0)
            == 2 * jax.lax.broadcasted_iota(jnp.int32, (W1 - 1, Wp), 1)
            ).astype(jnp.float32)
    p = jax.lax.dot_general(rpair, dsel, (((2,), (0,)), ((), ())),
                            preferred_element_type=jnp.float32, precision=jax.lax.Precision.HIGHEST)

    y2 = _conv3x3(p, w2_ref[...], b2_ref[...], a2_ref[...])
    y3 = _conv3x3(y2, w3_ref[...], b3_ref[...], a3_ref[...])

    logits = jax.lax.dot_general(w41_ref[...], y3, (((1,), (0,)), ((), ())),
                                 preferred_element_type=jnp.float32, precision=jax.lax.Precision.DEFAULT)
    logits = logits + b41_ref[...]
    prob = jax.nn.sigmoid(logits[1] - logits[0])
    reg = jax.lax.dot_general(w42_ref[...], y3, (((1,), (0,)), ((), ())),
                              preferred_element_type=jnp.float32, precision=jax.lax.Precision.DEFAULT)
    reg = reg + b42_ref[...]

    H3, W3 = prob.shape
    gy = jax.lax.broadcasted_iota(jnp.int32, (H3, W3), 0).astype(jnp.float32)
    gx = jax.lax.broadcasted_iota(jnp.int32, (H3, W3), 1).astype(jnp.float32)
    inv = 1.0 / sc
    x1 = (2.0 * gx + 1.0) * inv
    yy1 = (2.0 * gy + 1.0) * inv
    x2 = (2.0 * gx + 12.0) * inv
    yy2 = (2.0 * gy + 12.0) * inv
    bw = x2 - x1
    bh = yy2 - yy1
    bx1 = x1 + reg[0] * bw
    by1 = yy1 + reg[1] * bh
    bx2 = x2 + reg[2] * bw
    by2 = yy2 + reg[3] * bh

    score_ref[0] = prob
    box_ref[0] = jnp.stack([bx1, by1, bx2, by2], axis=0)


def _run_pnet_scale(idx, img, w1m, b1, a1, w2m, b2, a2, w3m, b3, a3,
                    w41m, b41, w42m, b42):
    s2 = _SIZES[idx]
    h3 = (s2 - 2) // 2 - 4
    B = img.shape[0]
    full = lambda arr: pl.BlockSpec(arr.shape, lambda b: (0,) * arr.ndim)
    kern = functools.partial(_pnet_scale_kernel, _SCALES[idx], s2)
    score, box = pl.pallas_call(
        kern,
        grid=(B,),
        in_specs=[
            pl.BlockSpec((1, 3, s2, s2), lambda b: (b, 0, 0, 0)),
            full(w1m), full(b1), full(a1), full(w2m), full(b2),
            full(a2), full(w3m), full(b3), full(a3), full(w41m), full(b41),
            full(w42m), full(b42),
        ],
        out_specs=[
            pl.BlockSpec((1, h3, h3), lambda b: (b, 0, 0)),
            pl.BlockSpec((1, 4, h3, h3), lambda b: (b, 0, 0, 0)),
        ],
        out_shape=[
            jax.ShapeDtypeStruct((B, h3, h3), jnp.float32),
            jax.ShapeDtypeStruct((B, 4, h3, h3), jnp.float32),
        ],
        name=f"pnet_s{idx}",
        compiler_params=pltpu.CompilerParams(dimension_semantics=("parallel",)),
    )(img, w1m, b1, a1, w2m, b2, a2, w3m, b3, a3, w41m, b41, w42m, b42)
    return score, box


def _topk_nms_kernel(s_ref, bx_ref, out_ref):
    B = s_ref.shape[0]
    shape = (46, 128)
    pos = (jax.lax.broadcasted_iota(jnp.int32, shape, 0) * 128
           + jax.lax.broadcasted_iota(jnp.int32, shape, 1))
    posf = pos.astype(jnp.float32)
    lane = jax.lax.broadcasted_iota(jnp.int32, (1, 128), 1)
    z128 = jnp.zeros((1, 128), jnp.float32)

    init = []
    for b in range(B):
        S = jnp.where(pos < _NTOT, s_ref[b], -1e30)
        init.append((S, z128, z128, z128, z128, z128))
    init = tuple(x for st in init for x in st)

    def body(k, carry):
        sts = [carry[6 * b:6 * b + 6] for b in range(B)]
        out = []
        lhot = lane == k
        for b in range(B):
            Sc, sx1, sy1, sx2, sy2, ss = sts[b]
            mflat = jnp.max(Sc)
            hit = Sc == mflat
            selp = jnp.min(jnp.where(hit, posf, 1e9))
            hsel = posf == selp
            hself = hsel.astype(jnp.float32)
            bx1 = jnp.sum(hself * bx_ref[b, 0])
            by1 = jnp.sum(hself * bx_ref[b, 1])
            bx2 = jnp.sum(hself * bx_ref[b, 2])
            by2 = jnp.sum(hself * bx_ref[b, 3])
            sx1 = jnp.where(lhot, bx1, sx1)
            sy1 = jnp.where(lhot, by1, sy1)
            sx2 = jnp.where(lhot, bx2, sx2)
            sy2 = jnp.where(lhot, by2, sy2)
            ss = jnp.where(lhot, mflat, ss)
            Sc = jnp.where(hsel, -1e30, Sc)
            out.append((Sc, sx1, sy1, sx2, sy2, ss))
        return tuple(x for st in out for x in st)

    fin = jax.lax.fori_loop(0, _K, body, init)

    sub = jax.lax.broadcasted_iota(jnp.int32, (128, 1), 0)
    nms_init = []
    iou_mats = []
    sorted_sts = []
    for b in range(B):
        _, sx1, sy1, sx2, sy2, ss = fin[6 * b:6 * b + 6]
        sorted_sts.append((sx1, sy1, sx2, sy2, ss))
        area_r = (sx2 - sx1) * (sy2 - sy1)
        x1c = sx1.T
        y1c = sy1.T
        x2c = sx2.T
        y2c = sy2.T
        area_c = (x2c - x1c) * (y2c - y1c)
        xx1 = jnp.maximum(x1c, sx1)
        yy1 = jnp.maximum(y1c, sy1)
        xx2 = jnp.minimum(x2c, sx2)
        yy2 = jnp.minimum(y2c, sy2)
        inter = jnp.maximum(0.0, xx2 - xx1) * jnp.maximum(0.0, yy2 - yy1)
        iou = inter / (area_c + area_r - inter + 1e-9)
        cmat = (iou > _NMS_T).astype(jnp.float32)
        iou_mats.append(cmat)
        nms_init.append((z128, z128))
    nms_init = tuple(x for st in nms_init for x in st)

    def nms_body(k, carry):
        sts = [carry[2 * b:2 * b + 2] for b in range(B)]
        out = []
        lhot = lane == k
        colhot = (sub == k).astype(jnp.float32)
        for b in range(B):
            kept, supv = sts[b]
            ss = sorted_sts[b][4]
            rowk = jnp.sum(iou_mats[b] * colhot, axis=0, keepdims=True)
            kept = jnp.where(lhot & (ss > _THRESH) & (supv < 0.5), 1.0, kept)
            keptk = jnp.max(kept * lhot.astype(jnp.float32))
            supv = jnp.where((rowk > 0.5) & (keptk > 0), 1.0, supv)
            out.append((kept, supv))
        return tuple(x for st in out for x in st)

    nfin = jax.lax.fori_loop(0, _K, nms_body, nms_init)
    for b in range(B):
        sx1, sy1, sx2, sy2, ss = sorted_sts[b]
        kept = nfin[2 * b]
        out_ref[b] = jnp.concatenate(
            [sx1, sy1, sx2, sy2, ss * kept, jnp.zeros((3, 128), jnp.float32)],
            axis=0)


def kernel(img, w1, b1, a1, w2, b2, a2, w3, b3, a3, w41, b41, w42, b42):
    B = img.shape[0]
    xn = (img - 127.5) / 128.0

    w1m = w1.transpose(0, 2, 3, 1).reshape(10, 27)
    w2m = w2.transpose(0, 2, 3, 1).reshape(16, 90)
    w3m = w3.transpose(0, 2, 3, 1).reshape(32, 144)
    w41m = w41.reshape(2, 32)
    w42m = w42.reshape(4, 32)
    b1r, a1r = b1.reshape(10, 1, 1), a1.reshape(10, 1, 1)
    b2r, a2r = b2.reshape(16, 1, 1), a2.reshape(16, 1, 1)
    b3r, a3r = b3.reshape(32, 1, 1), a3.reshape(32, 1, 1)
    b41r = b41.reshape(2, 1, 1)
    b42r = b42.reshape(4, 1, 1)

    scores = []
    boxes = []
    for i in range(3):
        im_i = jax.image.resize(xn, (B, 3, _SIZES[i], _SIZES[i]),
                                method='bilinear')
        s, bx = _run_pnet_scale(i, im_i, w1m, b1r, a1r, w2m, b2r, a2r,
                                w3m, b3r, a3r, w41m, b41r, w42m, b42r)
        scores.append(s.reshape(B, -1))
        boxes.append(bx.reshape(B, 4, -1))

    s_all = jnp.concatenate(scores, axis=1)
    b_all = jnp.concatenate(boxes, axis=2)
    s_pl = jnp.pad(s_all, ((0, 0), (0, _NPAD - _NTOT))).reshape(B, 46, 128)
    b_pl = jnp.pad(b_all, ((0, 0), (0, 0), (0, _NPAD - _NTOT))).reshape(B, 4, 46, 128)

    out = pl.pallas_call(
        _topk_nms_kernel,
        out_shape=jax.ShapeDtypeStruct((B, 8, 128), jnp.float32),
        name="topk_nms",
    )(s_pl, b_pl)

    res = out[:, :5, :_K].transpose(0, 2, 1)
    return res
